# baseline (speedup 1.0000x reference)
import functools

import jax
import jax.numpy as jnp
from jax import lax
from jax.experimental import pallas as pl
from jax.experimental.pallas import tpu as pltpu
from jax.experimental.pallas import tpu_sc as plsc

NC = 2
NS = 16
NW = NC * NS
LANES = 16
K = 64
NB = 3


def _rpt(n):
    return -(-(-(-n // NS)) // 8) * 8


def _sc_agg(n, nch, d):
    assert nch % (2 * NB) == 0
    rpt = n // NS
    assert rpt * NS == n and rpt % 8 == 0
    mesh = plsc.VectorSubcoreMesh(core_axis_name="c", subcore_axis_name="s", num_cores=NC, num_subcores=NS)

    def body(src_hbm, dst_hbm, y_hbm, out_hbm, sidx, didx, rows, gsems, ssems,
             isems, acc_sh):
        c = lax.axis_index("c")
        s = lax.axis_index("s")
        wid = c * NS + s
        row0 = s * rpt

        def idx_issue(ch, sl):
            pltpu.async_copy(src_hbm.at[wid, ch], sidx[sl], isems[sl])
            pltpu.async_copy(dst_hbm.at[wid, ch], didx[sl], isems[sl])

        def idx_wait(ch, sl):
            pltpu.make_async_copy(src_hbm.at[wid, ch], sidx[sl], isems[sl]).wait()
            pltpu.make_async_copy(dst_hbm.at[wid, ch], didx[sl], isems[sl]).wait()

        def gather_issue(sl, b):
            pltpu.async_copy(y_hbm.at[sidx[sl]], rows[b], gsems[b])

        def gather_wait(sl, b):
            pltpu.make_async_copy(y_hbm.at[sidx[sl]], rows[b], gsems[b]).wait()

        def scat_issue(sl, b):
            pltpu.async_copy(rows[b], acc_sh.at[didx[sl]], ssems[b], add=True)

        def scat_wait(sl, b):
            pltpu.make_async_copy(rows[b], acc_sh.at[didx[sl]], ssems[b]).wait()

        for u in range(2 * NB):
            idx_issue(u, u)
        pltpu.sync_copy(y_hbm.at[pl.ds(row0, rpt)], acc_sh.at[pl.ds(row0, rpt)])
        plsc.subcore_barrier()
        for b in range(NB):
            idx_wait(b, b)
            gather_issue(b, b)

        @pl.loop(0, nch, step=2 * NB)
        def _(i):
            for h in range(2):
                for b in range(NB):
                    gather_wait(h * NB + b, b)
                for b in range(NB):
                    scat_issue(h * NB + b, b)
                for b in range(NB):
                    scat_wait(h * NB + b, b)
                for b in range(NB):
                    sl = h * NB + b
                    nsl = (1 - h) * NB + b
                    refill = i + h * NB + b + 2 * NB
                    nxt = i + h * NB + b + NB

                    @pl.when(refill < nch)
                    def _():
                        idx_issue(refill, sl)

                    @pl.when(nxt < nch)
                    def _():
                        idx_wait(nxt, nsl)
                        gather_issue(nsl, b)

        plsc.subcore_barrier()
        pltpu.sync_copy(acc_sh.at[pl.ds(row0, rpt)], out_hbm.at[c, pl.ds(row0, rpt)])

    return pl.kernel(
        body,
        out_type=jax.ShapeDtypeStruct((NC, n, d), jnp.float32),
        mesh=mesh,
        scratch_types=[
            [pltpu.VMEM((K,), jnp.int32) for _ in range(2 * NB)],
            [pltpu.VMEM((K,), jnp.int32) for _ in range(2 * NB)],
            [pltpu.VMEM((K, d), jnp.float32) for _ in range(NB)],
            [pltpu.SemaphoreType.DMA for _ in range(NB)],
            [pltpu.SemaphoreType.DMA for _ in range(NB)],
            [pltpu.SemaphoreType.DMA for _ in range(2 * NB)],
            pltpu.VMEM_SHARED((n, d), jnp.float32),
        ],
    )


def _sc_deg(n, e):
    epw = e // NW
    KD = 80
    assert epw % KD == 0
    nch = epw // KD
    rpt = n // NS
    assert rpt * NS == n and rpt % 8 == 0
    mesh = plsc.VectorSubcoreMesh(core_axis_name="c", subcore_axis_name="s", num_cores=NC, num_subcores=NS)

    def body(dst_hbm, ones_hbm, out_hbm, dst_v, ones_v, acc_sh):
        c = lax.axis_index("c")
        s = lax.axis_index("s")
        wid = c * NS + s
        row0 = s * rpt

        @pl.loop(0, KD)
        def _(i):
            ones_v[i, :] = jnp.ones((LANES,), jnp.float32)

        pltpu.sync_copy(ones_hbm.at[pl.ds(row0, rpt)], acc_sh.at[pl.ds(row0, rpt)])
        plsc.subcore_barrier()
        ebase = wid * epw

        @pl.loop(0, nch)
        def _(i):
            off = ebase + i * KD
            pltpu.sync_copy(dst_hbm.at[pl.ds(off, KD)], dst_v)
            pltpu.sync_copy(ones_v, acc_sh.at[dst_v], add=True)

        plsc.subcore_barrier()
        pltpu.sync_copy(acc_sh.at[pl.ds(row0, rpt)], out_hbm.at[c, pl.ds(row0, rpt)])

    return pl.kernel(
        body,
        out_type=jax.ShapeDtypeStruct((NC, n, LANES), jnp.float32),
        mesh=mesh,
        scratch_types=[
            pltpu.VMEM((KD,), jnp.int32),
            pltpu.VMEM((KD, LANES), jnp.float32),
            pltpu.VMEM_SHARED((n, LANES), jnp.float32),
        ],
    )


BN = 2000


def _dinv_blk(deg_ref):
    d = deg_ref[0, :, 0] + deg_ref[1, :, 0] - 1.0
    return lax.rsqrt(d)[:, None]


def _tc_prologue(degp, x, w, npad):
    n, f = x.shape
    h = w.shape[1]

    def body(deg_ref, x_ref, w_ref, y_ref):
        y_ref[...] = _dinv_blk(deg_ref) * jnp.dot(
            x_ref[...], w_ref[...], preferred_element_type=jnp.float32)

    return pl.pallas_call(
        body,
        grid=(n // BN,),
        in_specs=[
            pl.BlockSpec((NC, BN, LANES), lambda i: (0, i, 0)),
            pl.BlockSpec((BN, f), lambda i: (i, 0)),
            pl.BlockSpec((f, h), lambda i: (0, 0)),
        ],
        out_specs=pl.BlockSpec((BN, h), lambda i: (i, 0)),
        out_shape=jax.ShapeDtypeStruct((npad, h), jnp.float32),
    )(degp, x, w)


def _tc_mid(degp, p, y, b, w, n):
    npad, h = y.shape
    h2 = w.shape[1]

    def body(deg_ref, p_ref, y_ref, b_ref, w_ref, out_ref):
        dinv = _dinv_blk(deg_ref)
        pre = dinv * (p_ref[0] + p_ref[1] - y_ref[...]) + b_ref[...]
        act = jnp.maximum(pre, 0.0)
        out_ref[...] = dinv * jnp.dot(
            act, w_ref[...], preferred_element_type=jnp.float32)

    return pl.pallas_call(
        body,
        grid=(n // BN,),
        in_specs=[
            pl.BlockSpec((NC, BN, LANES), lambda i: (0, i, 0)),
            pl.BlockSpec((NC, BN, h), lambda i: (0, i, 0)),
            pl.BlockSpec((BN, h), lambda i: (i, 0)),
            pl.BlockSpec((1, h), lambda i: (0, 0)),
            pl.BlockSpec((h, h2), lambda i: (0, 0)),
        ],
        out_specs=pl.BlockSpec((BN, h2), lambda i: (i, 0)),
        out_shape=jax.ShapeDtypeStruct((npad, h2), jnp.float32),
    )(degp, p, y, b, w)


def _tc_final(degp, q, y, b, wc, bc, n):
    npad, h = y.shape
    ncls = wc.shape[1]

    def body(deg_ref, q_ref, y_ref, b_ref, wc_ref, bc_ref, out_ref):
        dinv = _dinv_blk(deg_ref)
        pre = dinv * (q_ref[0] + q_ref[1] - y_ref[...]) + b_ref[...]
        act = jnp.maximum(pre, 0.0)
        logits = jnp.dot(act, wc_ref[...],
                         preferred_element_type=jnp.float32) + bc_ref[...]
        m = jnp.max(logits, axis=1, keepdims=True)
        lse = jnp.log(jnp.sum(jnp.exp(logits - m), axis=1, keepdims=True)) + m
        out_ref[...] = logits - lse

    return pl.pallas_call(
        body,
        grid=(n // BN,),
        in_specs=[
            pl.BlockSpec((NC, BN, LANES), lambda i: (0, i, 0)),
            pl.BlockSpec((NC, BN, h), lambda i: (0, i, 0)),
            pl.BlockSpec((BN, h), lambda i: (i, 0)),
            pl.BlockSpec((1, h), lambda i: (0, 0)),
            pl.BlockSpec((h, ncls), lambda i: (0, 0)),
            pl.BlockSpec((1, ncls), lambda i: (0, 0)),
        ],
        out_specs=pl.BlockSpec((BN, ncls), lambda i: (i, 0)),
        out_shape=jax.ShapeDtypeStruct((n, ncls), jnp.float32),
    )(degp, q, y, b, wc, bc)


def kernel(x, edge_idx, W1, b1, W2, b2, Wc, bc):
    n, f = x.shape
    e = edge_idx.shape[1]
    npad = _rpt(n) * NS
    src = edge_idx[0].astype(jnp.int32)
    dst = edge_idx[1].astype(jnp.int32)
    ones = jnp.ones((npad, LANES), jnp.float32)

    epw = e // NW
    epw_pad = -(-epw // (K * 2 * NB)) * (K * 2 * NB)
    nch = epw_pad // K
    padw = epw_pad - epw
    pad_rows = n + jnp.arange(NW * padw, dtype=jnp.int32).reshape(NW, padw) % (npad - n)
    src2 = jnp.concatenate([src.reshape(NW, epw), pad_rows], axis=1).reshape(NW, nch, K)
    dst2 = jnp.concatenate([dst.reshape(NW, epw), pad_rows], axis=1).reshape(NW, nch, K)

    degp = _sc_deg(npad, e)(dst, ones)
    y1 = _tc_prologue(degp, x, W1, npad)
    p = _sc_agg(npad, nch, W1.shape[1])(src2, dst2, y1)
    y2 = _tc_mid(degp, p, y1, b1.reshape(1, -1), W2, n)
    q = _sc_agg(npad, nch, W2.shape[1])(src2, dst2, y2)
    return _tc_final(degp, q, y2, b2.reshape(1, -1), Wc, bc.reshape(1, -1), n)


# speedup vs baseline: 1.0537x; 1.0537x over previous
import functools

import jax
import jax.numpy as jnp
from jax import lax
from jax.experimental import pallas as pl
from jax.experimental.pallas import tpu as pltpu
from jax.experimental.pallas import tpu_sc as plsc

NC = 2
NS = 16
NW = NC * NS
LANES = 16
K = 128
NB = 2


def _rpt(n):
    return -(-(-(-n // NS)) // 8) * 8


def _sc_agg(n, nch, d):
    assert nch % (2 * NB) == 0
    rpt = n // NS
    assert rpt * NS == n and rpt % 8 == 0
    mesh = plsc.VectorSubcoreMesh(core_axis_name="c", subcore_axis_name="s", num_cores=NC, num_subcores=NS)

    def body(src_hbm, dst_hbm, y_hbm, out_hbm, sidx, didx, rows, gsems, ssems,
             isems, acc_sh):
        c = lax.axis_index("c")
        s = lax.axis_index("s")
        wid = c * NS + s
        row0 = s * rpt

        def idx_issue(ch, sl):
            pltpu.async_copy(src_hbm.at[wid, ch], sidx[sl], isems[sl])
            pltpu.async_copy(dst_hbm.at[wid, ch], didx[sl], isems[sl])

        def idx_wait(ch, sl):
            pltpu.make_async_copy(src_hbm.at[wid, ch], sidx[sl], isems[sl]).wait()
            pltpu.make_async_copy(dst_hbm.at[wid, ch], didx[sl], isems[sl]).wait()

        def gather_issue(sl, b):
            pltpu.async_copy(y_hbm.at[sidx[sl]], rows[b], gsems[b])

        def gather_wait(sl, b):
            pltpu.make_async_copy(y_hbm.at[sidx[sl]], rows[b], gsems[b]).wait()

        def scat_issue(sl, b):
            pltpu.async_copy(rows[b], acc_sh.at[didx[sl]], ssems[b], add=True)

        def scat_wait(sl, b):
            pltpu.make_async_copy(rows[b], acc_sh.at[didx[sl]], ssems[b]).wait()

        for u in range(2 * NB):
            idx_issue(u, u)
        pltpu.sync_copy(y_hbm.at[pl.ds(row0, rpt)], acc_sh.at[pl.ds(row0, rpt)])
        plsc.subcore_barrier()
        for b in range(NB):
            idx_wait(b, b)
            gather_issue(b, b)

        @pl.loop(0, nch, step=2 * NB)
        def _(i):
            for h in range(2):
                for b in range(NB):
                    gather_wait(h * NB + b, b)
                for b in range(NB):
                    scat_issue(h * NB + b, b)
                for b in range(NB):
                    scat_wait(h * NB + b, b)
                for b in range(NB):
                    sl = h * NB + b
                    nsl = (1 - h) * NB + b
                    refill = i + h * NB + b + 2 * NB
                    nxt = i + h * NB + b + NB

                    @pl.when(refill < nch)
                    def _():
                        idx_issue(refill, sl)

                    @pl.when(nxt < nch)
                    def _():
                        idx_wait(nxt, nsl)
                        gather_issue(nsl, b)

        plsc.subcore_barrier()
        pltpu.sync_copy(acc_sh.at[pl.ds(row0, rpt)], out_hbm.at[c, pl.ds(row0, rpt)])

    return pl.kernel(
        body,
        out_type=jax.ShapeDtypeStruct((NC, n, d), jnp.float32),
        mesh=mesh,
        scratch_types=[
            [pltpu.VMEM((K,), jnp.int32) for _ in range(2 * NB)],
            [pltpu.VMEM((K,), jnp.int32) for _ in range(2 * NB)],
            [pltpu.VMEM((K, d), jnp.float32) for _ in range(NB)],
            [pltpu.SemaphoreType.DMA for _ in range(NB)],
            [pltpu.SemaphoreType.DMA for _ in range(NB)],
            [pltpu.SemaphoreType.DMA for _ in range(2 * NB)],
            pltpu.VMEM_SHARED((n, d), jnp.float32),
        ],
    )


def _sc_deg(n, e):
    epw = e // NW
    KD = 80
    assert epw % KD == 0
    nch = epw // KD
    rpt = n // NS
    assert rpt * NS == n and rpt % 8 == 0
    mesh = plsc.VectorSubcoreMesh(core_axis_name="c", subcore_axis_name="s", num_cores=NC, num_subcores=NS)

    def body(dst_hbm, ones_hbm, out_hbm, dst_v, ones_v, acc_sh):
        c = lax.axis_index("c")
        s = lax.axis_index("s")
        wid = c * NS + s
        row0 = s * rpt

        @pl.loop(0, KD)
        def _(i):
            ones_v[i, :] = jnp.ones((LANES,), jnp.float32)

        pltpu.sync_copy(ones_hbm.at[pl.ds(row0, rpt)], acc_sh.at[pl.ds(row0, rpt)])
        plsc.subcore_barrier()
        ebase = wid * epw

        @pl.loop(0, nch)
        def _(i):
            off = ebase + i * KD
            pltpu.sync_copy(dst_hbm.at[pl.ds(off, KD)], dst_v)
            pltpu.sync_copy(ones_v, acc_sh.at[dst_v], add=True)

        plsc.subcore_barrier()
        pltpu.sync_copy(acc_sh.at[pl.ds(row0, rpt)], out_hbm.at[c, pl.ds(row0, rpt)])

    return pl.kernel(
        body,
        out_type=jax.ShapeDtypeStruct((NC, n, LANES), jnp.float32),
        mesh=mesh,
        scratch_types=[
            pltpu.VMEM((KD,), jnp.int32),
            pltpu.VMEM((KD, LANES), jnp.float32),
            pltpu.VMEM_SHARED((n, LANES), jnp.float32),
        ],
    )


BN = 2000


def _dinv_blk(deg_ref):
    d = deg_ref[0, :, 0] + deg_ref[1, :, 0] - 1.0
    return lax.rsqrt(d)[:, None]


def _tc_prologue(degp, x, w, npad):
    n, f = x.shape
    h = w.shape[1]

    def body(deg_ref, x_ref, w_ref, y_ref):
        y_ref[...] = _dinv_blk(deg_ref) * jnp.dot(
            x_ref[...], w_ref[...], preferred_element_type=jnp.float32)

    return pl.pallas_call(
        body,
        grid=(n // BN,),
        in_specs=[
            pl.BlockSpec((NC, BN, LANES), lambda i: (0, i, 0)),
            pl.BlockSpec((BN, f), lambda i: (i, 0)),
            pl.BlockSpec((f, h), lambda i: (0, 0)),
        ],
        out_specs=pl.BlockSpec((BN, h), lambda i: (i, 0)),
        out_shape=jax.ShapeDtypeStruct((npad, h), jnp.float32),
    )(degp, x, w)


def _tc_mid(degp, p, y, b, w, n):
    npad, h = y.shape
    h2 = w.shape[1]

    def body(deg_ref, p_ref, y_ref, b_ref, w_ref, out_ref):
        dinv = _dinv_blk(deg_ref)
        pre = dinv * (p_ref[0] + p_ref[1] - y_ref[...]) + b_ref[...]
        act = jnp.maximum(pre, 0.0)
        out_ref[...] = dinv * jnp.dot(
            act, w_ref[...], preferred_element_type=jnp.float32)

    return pl.pallas_call(
        body,
        grid=(n // BN,),
        in_specs=[
            pl.BlockSpec((NC, BN, LANES), lambda i: (0, i, 0)),
            pl.BlockSpec((NC, BN, h), lambda i: (0, i, 0)),
            pl.BlockSpec((BN, h), lambda i: (i, 0)),
            pl.BlockSpec((1, h), lambda i: (0, 0)),
            pl.BlockSpec((h, h2), lambda i: (0, 0)),
        ],
        out_specs=pl.BlockSpec((BN, h2), lambda i: (i, 0)),
        out_shape=jax.ShapeDtypeStruct((npad, h2), jnp.float32),
    )(degp, p, y, b, w)


def _tc_final(degp, q, y, b, wc, bc, n):
    npad, h = y.shape
    ncls = wc.shape[1]

    def body(deg_ref, q_ref, y_ref, b_ref, wc_ref, bc_ref, out_ref):
        dinv = _dinv_blk(deg_ref)
        pre = dinv * (q_ref[0] + q_ref[1] - y_ref[...]) + b_ref[...]
        act = jnp.maximum(pre, 0.0)
        logits = jnp.dot(act, wc_ref[...],
                         preferred_element_type=jnp.float32) + bc_ref[...]
        m = jnp.max(logits, axis=1, keepdims=True)
        lse = jnp.log(jnp.sum(jnp.exp(logits - m), axis=1, keepdims=True)) + m
        out_ref[...] = logits - lse

    return pl.pallas_call(
        body,
        grid=(n // BN,),
        in_specs=[
            pl.BlockSpec((NC, BN, LANES), lambda i: (0, i, 0)),
            pl.BlockSpec((NC, BN, h), lambda i: (0, i, 0)),
            pl.BlockSpec((BN, h), lambda i: (i, 0)),
            pl.BlockSpec((1, h), lambda i: (0, 0)),
            pl.BlockSpec((h, ncls), lambda i: (0, 0)),
            pl.BlockSpec((1, ncls), lambda i: (0, 0)),
        ],
        out_specs=pl.BlockSpec((BN, ncls), lambda i: (i, 0)),
        out_shape=jax.ShapeDtypeStruct((n, ncls), jnp.float32),
    )(degp, q, y, b, wc, bc)


def kernel(x, edge_idx, W1, b1, W2, b2, Wc, bc):
    n, f = x.shape
    e = edge_idx.shape[1]
    npad = _rpt(n) * NS
    src = edge_idx[0].astype(jnp.int32)
    dst = edge_idx[1].astype(jnp.int32)
    ones = jnp.ones((npad, LANES), jnp.float32)

    epw = e // NW
    epw_pad = -(-epw // (K * 2 * NB)) * (K * 2 * NB)
    nch = epw_pad // K
    padw = epw_pad - epw
    pad_rows = n + jnp.arange(NW * padw, dtype=jnp.int32).reshape(NW, padw) % (npad - n)
    src2 = jnp.concatenate([src.reshape(NW, epw), pad_rows], axis=1).reshape(NW, nch, K)
    dst2 = jnp.concatenate([dst.reshape(NW, epw), pad_rows], axis=1).reshape(NW, nch, K)

    degp = _sc_deg(npad, e)(dst, ones)
    y1 = _tc_prologue(degp, x, W1, npad)
    p = _sc_agg(npad, nch, W1.shape[1])(src2, dst2, y1)
    y2 = _tc_mid(degp, p, y1, b1.reshape(1, -1), W2, n)
    q = _sc_agg(npad, nch, W2.shape[1])(src2, dst2, y2)
    return _tc_final(degp, q, y2, b2.reshape(1, -1), Wc, bc.reshape(1, -1), n)


# speedup vs baseline: 1.1867x; 1.1262x over previous
import functools

import jax
import jax.numpy as jnp
from jax import lax
from jax.experimental import pallas as pl
from jax.experimental.pallas import tpu as pltpu
from jax.experimental.pallas import tpu_sc as plsc

NC = 2
NS = 16
NW = NC * NS
LANES = 16
K = 128
NB = 2


def _rpt(n):
    return -(-(-(-n // NS)) // 8) * 8


def _sc_agg(n, nch, d):
    assert nch % (2 * NB) == 0
    rpt = n // NS
    assert rpt * NS == n and rpt % 8 == 0
    mesh = plsc.VectorSubcoreMesh(core_axis_name="c", subcore_axis_name="s", num_cores=NC, num_subcores=NS)

    def body(src_hbm, dst_hbm, y_hbm, out_hbm, sidx, didx, rows, gsems, ssems,
             isems, acc_sh):
        c = lax.axis_index("c")
        s = lax.axis_index("s")
        wid = c * NS + s
        row0 = s * rpt

        def idx_issue(ch, sl):
            pltpu.async_copy(src_hbm.at[wid, ch], sidx[sl], isems[sl])
            pltpu.async_copy(dst_hbm.at[wid, ch], didx[sl], isems[sl])

        def idx_wait(ch, sl):
            pltpu.make_async_copy(src_hbm.at[wid, ch], sidx[sl], isems[sl]).wait()
            pltpu.make_async_copy(dst_hbm.at[wid, ch], didx[sl], isems[sl]).wait()

        def gather_issue(sl, b):
            pltpu.async_copy(y_hbm.at[sidx[sl]], rows[b], gsems[b])

        def gather_wait(sl, b):
            pltpu.make_async_copy(y_hbm.at[sidx[sl]], rows[b], gsems[b]).wait()

        def scat_issue(sl, b):
            pltpu.async_copy(rows[b], acc_sh.at[didx[sl]], ssems[b], add=True)

        def scat_wait(sl, b):
            pltpu.make_async_copy(rows[b], acc_sh.at[didx[sl]], ssems[b]).wait()

        for u in range(2 * NB):
            idx_issue(u, u)
        pltpu.sync_copy(y_hbm.at[pl.ds(row0, rpt)], acc_sh.at[pl.ds(row0, rpt)])
        plsc.subcore_barrier()
        for b in range(NB):
            idx_wait(b, b)
            gather_issue(b, b)

        @pl.loop(0, nch, step=2 * NB)
        def _(i):
            for h in range(2):
                for b in range(NB):
                    gather_wait(h * NB + b, b)
                for b in range(NB):
                    scat_issue(h * NB + b, b)
                for b in range(NB):
                    scat_wait(h * NB + b, b)
                for b in range(NB):
                    sl = h * NB + b
                    nsl = (1 - h) * NB + b
                    refill = i + h * NB + b + 2 * NB
                    nxt = i + h * NB + b + NB

                    @pl.when(refill < nch)
                    def _():
                        idx_issue(refill, sl)

                    @pl.when(nxt < nch)
                    def _():
                        idx_wait(nxt, nsl)
                        gather_issue(nsl, b)

        plsc.subcore_barrier()
        pltpu.sync_copy(acc_sh.at[pl.ds(row0, rpt)], out_hbm.at[c, pl.ds(row0, rpt)])

    return pl.kernel(
        body,
        out_type=jax.ShapeDtypeStruct((NC, n, d), jnp.float32),
        mesh=mesh,
        scratch_types=[
            [pltpu.VMEM((K,), jnp.int32) for _ in range(2 * NB)],
            [pltpu.VMEM((K,), jnp.int32) for _ in range(2 * NB)],
            [pltpu.VMEM((K, d), jnp.float32) for _ in range(NB)],
            [pltpu.SemaphoreType.DMA for _ in range(NB)],
            [pltpu.SemaphoreType.DMA for _ in range(NB)],
            [pltpu.SemaphoreType.DMA for _ in range(2 * NB)],
            pltpu.VMEM_SHARED((n, d), jnp.float32),
        ],
    )


def _sc_deg(n, nch):
    ND = 4
    assert nch % ND == 0
    rpt = n // NS
    assert rpt * NS == n and rpt % 8 == 0
    mesh = plsc.VectorSubcoreMesh(core_axis_name="c", subcore_axis_name="s", num_cores=NC, num_subcores=NS)

    def body(dst_hbm, ones_hbm, out_hbm, didx, ones_v, ssems, isems, acc_sh):
        c = lax.axis_index("c")
        s = lax.axis_index("s")
        wid = c * NS + s
        row0 = s * rpt

        @pl.loop(0, K)
        def _(i):
            ones_v[i, :] = jnp.ones((LANES,), jnp.float32)

        def idx_issue(ch, sl):
            pltpu.async_copy(dst_hbm.at[wid, ch], didx[sl], isems[sl])

        def idx_wait(ch, sl):
            pltpu.make_async_copy(dst_hbm.at[wid, ch], didx[sl], isems[sl]).wait()

        for u in range(ND):
            idx_issue(u, u)
        pltpu.sync_copy(ones_hbm.at[pl.ds(row0, rpt)], acc_sh.at[pl.ds(row0, rpt)])
        plsc.subcore_barrier()

        @pl.loop(0, nch, step=ND)
        def _(i):
            for b in range(ND):
                idx_wait(i + b, b)
                pltpu.async_copy(ones_v, acc_sh.at[didx[b]], ssems[b], add=True)
            for b in range(ND):
                pltpu.make_async_copy(ones_v, acc_sh.at[didx[b]], ssems[b]).wait()

                @pl.when(i + ND + b < nch)
                def _():
                    idx_issue(i + ND + b, b)

        plsc.subcore_barrier()
        pltpu.sync_copy(acc_sh.at[pl.ds(row0, rpt)], out_hbm.at[c, pl.ds(row0, rpt)])

    return pl.kernel(
        body,
        out_type=jax.ShapeDtypeStruct((NC, n, LANES), jnp.float32),
        mesh=mesh,
        scratch_types=[
            [pltpu.VMEM((K,), jnp.int32) for _ in range(ND)],
            pltpu.VMEM((K, LANES), jnp.float32),
            [pltpu.SemaphoreType.DMA for _ in range(ND)],
            [pltpu.SemaphoreType.DMA for _ in range(ND)],
            pltpu.VMEM_SHARED((n, LANES), jnp.float32),
        ],
    )


BN = 2000


def _dinv_blk(deg_ref):
    d = deg_ref[0, :, 0] + deg_ref[1, :, 0] - 1.0
    return lax.rsqrt(d)[:, None]


def _tc_matmul(x, w, npad):
    n, f = x.shape
    h = w.shape[1]

    def body(x_ref, w_ref, y_ref):
        y_ref[...] = jnp.dot(x_ref[...], w_ref[...],
                             preferred_element_type=jnp.float32)

    return pl.pallas_call(
        body,
        grid=(n // BN,),
        in_specs=[
            pl.BlockSpec((BN, f), lambda i: (i, 0)),
            pl.BlockSpec((f, h), lambda i: (0, 0)),
        ],
        out_specs=pl.BlockSpec((BN, h), lambda i: (i, 0)),
        out_shape=jax.ShapeDtypeStruct((npad, h), jnp.float32),
    )(x, w)


def _tc_scale(degp, xw, n):
    npad, h = xw.shape

    def body(deg_ref, xw_ref, y_ref):
        y_ref[...] = _dinv_blk(deg_ref) * xw_ref[...]

    return pl.pallas_call(
        body,
        grid=(n // BN,),
        in_specs=[
            pl.BlockSpec((NC, BN, LANES), lambda i: (0, i, 0)),
            pl.BlockSpec((BN, h), lambda i: (i, 0)),
        ],
        out_specs=pl.BlockSpec((BN, h), lambda i: (i, 0)),
        out_shape=jax.ShapeDtypeStruct((npad, h), jnp.float32),
    )(degp, xw)


def _tc_mid(degp, p, y, b, w, n):
    npad, h = y.shape
    h2 = w.shape[1]

    def body(deg_ref, p_ref, y_ref, b_ref, w_ref, out_ref):
        dinv = _dinv_blk(deg_ref)
        pre = dinv * (p_ref[0] + p_ref[1] - y_ref[...]) + b_ref[...]
        act = jnp.maximum(pre, 0.0)
        out_ref[...] = dinv * jnp.dot(
            act, w_ref[...], preferred_element_type=jnp.float32)

    return pl.pallas_call(
        body,
        grid=(n // BN,),
        in_specs=[
            pl.BlockSpec((NC, BN, LANES), lambda i: (0, i, 0)),
            pl.BlockSpec((NC, BN, h), lambda i: (0, i, 0)),
            pl.BlockSpec((BN, h), lambda i: (i, 0)),
            pl.BlockSpec((1, h), lambda i: (0, 0)),
            pl.BlockSpec((h, h2), lambda i: (0, 0)),
        ],
        out_specs=pl.BlockSpec((BN, h2), lambda i: (i, 0)),
        out_shape=jax.ShapeDtypeStruct((npad, h2), jnp.float32),
    )(degp, p, y, b, w)


def _tc_final(degp, q, y, b, wc, bc, n):
    npad, h = y.shape
    ncls = wc.shape[1]

    def body(deg_ref, q_ref, y_ref, b_ref, wc_ref, bc_ref, out_ref):
        dinv = _dinv_blk(deg_ref)
        pre = dinv * (q_ref[0] + q_ref[1] - y_ref[...]) + b_ref[...]
        act = jnp.maximum(pre, 0.0)
        logits = jnp.dot(act, wc_ref[...],
                         preferred_element_type=jnp.float32) + bc_ref[...]
        m = jnp.max(logits, axis=1, keepdims=True)
        lse = jnp.log(jnp.sum(jnp.exp(logits - m), axis=1, keepdims=True)) + m
        out_ref[...] = logits - lse

    return pl.pallas_call(
        body,
        grid=(n // BN,),
        in_specs=[
            pl.BlockSpec((NC, BN, LANES), lambda i: (0, i, 0)),
            pl.BlockSpec((NC, BN, h), lambda i: (0, i, 0)),
            pl.BlockSpec((BN, h), lambda i: (i, 0)),
            pl.BlockSpec((1, h), lambda i: (0, 0)),
            pl.BlockSpec((h, ncls), lambda i: (0, 0)),
            pl.BlockSpec((1, ncls), lambda i: (0, 0)),
        ],
        out_specs=pl.BlockSpec((BN, ncls), lambda i: (i, 0)),
        out_shape=jax.ShapeDtypeStruct((n, ncls), jnp.float32),
    )(degp, q, y, b, wc, bc)


def kernel(x, edge_idx, W1, b1, W2, b2, Wc, bc):
    n, f = x.shape
    e = edge_idx.shape[1]
    npad = _rpt(n) * NS
    src = edge_idx[0].astype(jnp.int32)
    dst = edge_idx[1].astype(jnp.int32)
    ones = jnp.ones((npad, LANES), jnp.float32)

    epw = e // NW
    epw_pad = -(-epw // (K * 2 * NB)) * (K * 2 * NB)
    nch = epw_pad // K
    padw = epw_pad - epw
    pad_rows = n + jnp.arange(NW * padw, dtype=jnp.int32).reshape(NW, padw) % (npad - n)
    src2 = jnp.concatenate([src.reshape(NW, epw), pad_rows], axis=1).reshape(NW, nch, K)
    dst2 = jnp.concatenate([dst.reshape(NW, epw), pad_rows], axis=1).reshape(NW, nch, K)

    degp = _sc_deg(npad, nch)(dst2, ones)
    xw1 = _tc_matmul(x, W1, npad)
    y1 = _tc_scale(degp, xw1, n)
    p = _sc_agg(npad, nch, W1.shape[1])(src2, dst2, y1)
    y2 = _tc_mid(degp, p, y1, b1.reshape(1, -1), W2, n)
    q = _sc_agg(npad, nch, W2.shape[1])(src2, dst2, y2)
    return _tc_final(degp, q, y2, b2.reshape(1, -1), Wc, bc.reshape(1, -1), n)


# speedup vs baseline: 1.1915x; 1.0041x over previous
import functools

import jax
import jax.numpy as jnp
from jax import lax
from jax.experimental import pallas as pl
from jax.experimental.pallas import tpu as pltpu
from jax.experimental.pallas import tpu_sc as plsc

NC = 2
NS = 16
NW = NC * NS
LANES = 16
K = 128
NB = 2


def _rpt(n):
    return -(-(-(-n // NS)) // 8) * 8


def _sc_agg(n, nch, d):
    assert nch % (2 * NB) == 0
    rpt = n // NS
    assert rpt * NS == n and rpt % 8 == 0
    mesh = plsc.VectorSubcoreMesh(core_axis_name="c", subcore_axis_name="s", num_cores=NC, num_subcores=NS)

    def body(src_hbm, dst_hbm, y_hbm, out_hbm, sidx, didx, rows, gsems, ssems,
             isems, acc_sh):
        c = lax.axis_index("c")
        s = lax.axis_index("s")
        wid = c * NS + s
        row0 = s * rpt

        def idx_issue(ch, sl):
            pltpu.async_copy(src_hbm.at[wid, ch], sidx[sl], isems[sl])
            pltpu.async_copy(dst_hbm.at[wid, ch], didx[sl], isems[sl])

        def idx_wait(ch, sl):
            pltpu.make_async_copy(src_hbm.at[wid, ch], sidx[sl], isems[sl]).wait()
            pltpu.make_async_copy(dst_hbm.at[wid, ch], didx[sl], isems[sl]).wait()

        def gather_issue(sl, b):
            pltpu.async_copy(y_hbm.at[sidx[sl]], rows[b], gsems[b])

        def gather_wait(sl, b):
            pltpu.make_async_copy(y_hbm.at[sidx[sl]], rows[b], gsems[b]).wait()

        def scat_issue(sl, b):
            pltpu.async_copy(rows[b], acc_sh.at[didx[sl]], ssems[b], add=True)

        def scat_wait(sl, b):
            pltpu.make_async_copy(rows[b], acc_sh.at[didx[sl]], ssems[b]).wait()

        for u in range(2 * NB):
            idx_issue(u, u)
        pltpu.sync_copy(y_hbm.at[pl.ds(row0, rpt)], acc_sh.at[pl.ds(row0, rpt)])
        plsc.subcore_barrier()
        for b in range(NB):
            idx_wait(b, b)
            gather_issue(b, b)

        @pl.loop(0, nch, step=2 * NB)
        def _(i):
            for h in range(2):
                for b in range(NB):
                    sl = h * NB + b
                    gather_wait(sl, b)
                    scat_issue(sl, b)
                    scat_wait(sl, b)
                for b in range(NB):
                    sl = h * NB + b
                    nsl = (1 - h) * NB + b
                    refill = i + h * NB + b + 2 * NB
                    nxt = i + h * NB + b + NB

                    @pl.when(refill < nch)
                    def _():
                        idx_issue(refill, sl)

                    @pl.when(nxt < nch)
                    def _():
                        idx_wait(nxt, nsl)
                        gather_issue(nsl, b)

        plsc.subcore_barrier()
        pltpu.sync_copy(acc_sh.at[pl.ds(row0, rpt)], out_hbm.at[c, pl.ds(row0, rpt)])

    return pl.kernel(
        body,
        out_type=jax.ShapeDtypeStruct((NC, n, d), jnp.float32),
        mesh=mesh,
        scratch_types=[
            [pltpu.VMEM((K,), jnp.int32) for _ in range(2 * NB)],
            [pltpu.VMEM((K,), jnp.int32) for _ in range(2 * NB)],
            [pltpu.VMEM((K, d), jnp.float32) for _ in range(NB)],
            [pltpu.SemaphoreType.DMA for _ in range(NB)],
            [pltpu.SemaphoreType.DMA for _ in range(NB)],
            [pltpu.SemaphoreType.DMA for _ in range(2 * NB)],
            pltpu.VMEM_SHARED((n, d), jnp.float32),
        ],
    )


def _sc_deg(n, nch):
    ND = 4
    assert nch % ND == 0
    rpt = n // NS
    assert rpt * NS == n and rpt % 8 == 0
    mesh = plsc.VectorSubcoreMesh(core_axis_name="c", subcore_axis_name="s", num_cores=NC, num_subcores=NS)

    def body(dst_hbm, ones_hbm, out_hbm, didx, ones_v, ssems, isems, acc_sh):
        c = lax.axis_index("c")
        s = lax.axis_index("s")
        wid = c * NS + s
        row0 = s * rpt

        @pl.loop(0, K)
        def _(i):
            ones_v[i, :] = jnp.ones((LANES,), jnp.float32)

        def idx_issue(ch, sl):
            pltpu.async_copy(dst_hbm.at[wid, ch], didx[sl], isems[sl])

        def idx_wait(ch, sl):
            pltpu.make_async_copy(dst_hbm.at[wid, ch], didx[sl], isems[sl]).wait()

        for u in range(ND):
            idx_issue(u, u)
        pltpu.sync_copy(ones_hbm.at[pl.ds(row0, rpt)], acc_sh.at[pl.ds(row0, rpt)])
        plsc.subcore_barrier()

        @pl.loop(0, nch, step=ND)
        def _(i):
            for b in range(ND):
                idx_wait(i + b, b)
                pltpu.async_copy(ones_v, acc_sh.at[didx[b]], ssems[b], add=True)
            for b in range(ND):
                pltpu.make_async_copy(ones_v, acc_sh.at[didx[b]], ssems[b]).wait()

                @pl.when(i + ND + b < nch)
                def _():
                    idx_issue(i + ND + b, b)

        plsc.subcore_barrier()
        pltpu.sync_copy(acc_sh.at[pl.ds(row0, rpt)], out_hbm.at[c, pl.ds(row0, rpt)])

    return pl.kernel(
        body,
        out_type=jax.ShapeDtypeStruct((NC, n, LANES), jnp.float32),
        mesh=mesh,
        scratch_types=[
            [pltpu.VMEM((K,), jnp.int32) for _ in range(ND)],
            pltpu.VMEM((K, LANES), jnp.float32),
            [pltpu.SemaphoreType.DMA for _ in range(ND)],
            [pltpu.SemaphoreType.DMA for _ in range(ND)],
            pltpu.VMEM_SHARED((n, LANES), jnp.float32),
        ],
    )


BN = 2000


def _dinv_blk(deg_ref):
    d = deg_ref[0, :, 0] + deg_ref[1, :, 0] - 1.0
    return lax.rsqrt(d)[:, None]


def _tc_matmul(x, w, npad):
    n, f = x.shape
    h = w.shape[1]

    def body(x_ref, w_ref, y_ref):
        y_ref[...] = jnp.dot(x_ref[...], w_ref[...],
                             preferred_element_type=jnp.float32)

    return pl.pallas_call(
        body,
        grid=(n // BN,),
        in_specs=[
            pl.BlockSpec((BN, f), lambda i: (i, 0)),
            pl.BlockSpec((f, h), lambda i: (0, 0)),
        ],
        out_specs=pl.BlockSpec((BN, h), lambda i: (i, 0)),
        out_shape=jax.ShapeDtypeStruct((npad, h), jnp.float32),
    )(x, w)


def _tc_scale(degp, xw, n):
    npad, h = xw.shape

    def body(deg_ref, xw_ref, y_ref):
        y_ref[...] = _dinv_blk(deg_ref) * xw_ref[...]

    return pl.pallas_call(
        body,
        grid=(n // BN,),
        in_specs=[
            pl.BlockSpec((NC, BN, LANES), lambda i: (0, i, 0)),
            pl.BlockSpec((BN, h), lambda i: (i, 0)),
        ],
        out_specs=pl.BlockSpec((BN, h), lambda i: (i, 0)),
        out_shape=jax.ShapeDtypeStruct((npad, h), jnp.float32),
    )(degp, xw)


def _tc_mid(degp, p, y, b, w, n):
    npad, h = y.shape
    h2 = w.shape[1]

    def body(deg_ref, p_ref, y_ref, b_ref, w_ref, out_ref):
        dinv = _dinv_blk(deg_ref)
        pre = dinv * (p_ref[0] + p_ref[1] - y_ref[...]) + b_ref[...]
        act = jnp.maximum(pre, 0.0)
        out_ref[...] = dinv * jnp.dot(
            act, w_ref[...], preferred_element_type=jnp.float32)

    return pl.pallas_call(
        body,
        grid=(n // BN,),
        in_specs=[
            pl.BlockSpec((NC, BN, LANES), lambda i: (0, i, 0)),
            pl.BlockSpec((NC, BN, h), lambda i: (0, i, 0)),
            pl.BlockSpec((BN, h), lambda i: (i, 0)),
            pl.BlockSpec((1, h), lambda i: (0, 0)),
            pl.BlockSpec((h, h2), lambda i: (0, 0)),
        ],
        out_specs=pl.BlockSpec((BN, h2), lambda i: (i, 0)),
        out_shape=jax.ShapeDtypeStruct((npad, h2), jnp.float32),
    )(degp, p, y, b, w)


def _tc_final(degp, q, y, b, wc, bc, n):
    npad, h = y.shape
    ncls = wc.shape[1]

    def body(deg_ref, q_ref, y_ref, b_ref, wc_ref, bc_ref, out_ref):
        dinv = _dinv_blk(deg_ref)
        pre = dinv * (q_ref[0] + q_ref[1] - y_ref[...]) + b_ref[...]
        act = jnp.maximum(pre, 0.0)
        logits = jnp.dot(act, wc_ref[...],
                         preferred_element_type=jnp.float32) + bc_ref[...]
        m = jnp.max(logits, axis=1, keepdims=True)
        lse = jnp.log(jnp.sum(jnp.exp(logits - m), axis=1, keepdims=True)) + m
        out_ref[...] = logits - lse

    return pl.pallas_call(
        body,
        grid=(n // BN,),
        in_specs=[
            pl.BlockSpec((NC, BN, LANES), lambda i: (0, i, 0)),
            pl.BlockSpec((NC, BN, h), lambda i: (0, i, 0)),
            pl.BlockSpec((BN, h), lambda i: (i, 0)),
            pl.BlockSpec((1, h), lambda i: (0, 0)),
            pl.BlockSpec((h, ncls), lambda i: (0, 0)),
            pl.BlockSpec((1, ncls), lambda i: (0, 0)),
        ],
        out_specs=pl.BlockSpec((BN, ncls), lambda i: (i, 0)),
        out_shape=jax.ShapeDtypeStruct((n, ncls), jnp.float32),
    )(degp, q, y, b, wc, bc)


def kernel(x, edge_idx, W1, b1, W2, b2, Wc, bc):
    n, f = x.shape
    e = edge_idx.shape[1]
    npad = _rpt(n) * NS
    src = edge_idx[0].astype(jnp.int32)
    dst = edge_idx[1].astype(jnp.int32)
    ones = jnp.ones((npad, LANES), jnp.float32)

    epw = e // NW
    epw_pad = -(-epw // (K * 2 * NB)) * (K * 2 * NB)
    nch = epw_pad // K
    padw = epw_pad - epw
    pad_rows = n + jnp.arange(NW * padw, dtype=jnp.int32).reshape(NW, padw) % (npad - n)
    src2 = jnp.concatenate([src.reshape(NW, epw), pad_rows], axis=1).reshape(NW, nch, K)
    dst2 = jnp.concatenate([dst.reshape(NW, epw), pad_rows], axis=1).reshape(NW, nch, K)

    degp = _sc_deg(npad, nch)(dst2, ones)
    xw1 = _tc_matmul(x, W1, npad)
    y1 = _tc_scale(degp, xw1, n)
    p = _sc_agg(npad, nch, W1.shape[1])(src2, dst2, y1)
    y2 = _tc_mid(degp, p, y1, b1.reshape(1, -1), W2, n)
    q = _sc_agg(npad, nch, W2.shape[1])(src2, dst2, y2)
    return _tc_final(degp, q, y2, b2.reshape(1, -1), Wc, bc.reshape(1, -1), n)


# speedup vs baseline: 1.2222x; 1.0258x over previous
import functools

import jax
import jax.numpy as jnp
from jax import lax
from jax.experimental import pallas as pl
from jax.experimental.pallas import tpu as pltpu
from jax.experimental.pallas import tpu_sc as plsc

NC = 2
NS = 16
NW = NC * NS
LANES = 16
K = 128
NB = 2


def _rpt(n):
    return -(-(-(-n // NS)) // 8) * 8


def _sc_agg(n, nch, d):
    assert nch % (2 * NB) == 0
    rpt = n // NS
    assert rpt * NS == n and rpt % 8 == 0
    mesh = plsc.VectorSubcoreMesh(core_axis_name="c", subcore_axis_name="s", num_cores=NC, num_subcores=NS)

    def body(src_hbm, dst_hbm, y_hbm, out_hbm, sidx, didx, rows, gsems, ssems,
             isems, acc_sh):
        c = lax.axis_index("c")
        s = lax.axis_index("s")
        wid = c * NS + s
        row0 = s * rpt

        def idx_issue(ch, sl):
            pltpu.async_copy(src_hbm.at[wid, ch], sidx[sl], isems[sl])
            pltpu.async_copy(dst_hbm.at[wid, ch], didx[sl], isems[sl])

        def idx_wait(ch, sl):
            pltpu.make_async_copy(src_hbm.at[wid, ch], sidx[sl], isems[sl]).wait()
            pltpu.make_async_copy(dst_hbm.at[wid, ch], didx[sl], isems[sl]).wait()

        def gather_issue(sl, b):
            pltpu.async_copy(y_hbm.at[sidx[sl]], rows[b], gsems[b])

        def gather_wait(sl, b):
            pltpu.make_async_copy(y_hbm.at[sidx[sl]], rows[b], gsems[b]).wait()

        def scat_issue(sl, b):
            pltpu.async_copy(rows[b], acc_sh.at[didx[sl]], ssems[b], add=True,
                             priority=1)

        def scat_wait(sl, b):
            pltpu.make_async_copy(rows[b], acc_sh.at[didx[sl]], ssems[b]).wait()

        for u in range(2 * NB):
            idx_issue(u, u)
        pltpu.sync_copy(y_hbm.at[pl.ds(row0, rpt)], acc_sh.at[pl.ds(row0, rpt)])
        plsc.subcore_barrier()
        for b in range(NB):
            idx_wait(b, b)
            gather_issue(b, b)

        @pl.loop(0, nch, step=2 * NB)
        def _(i):
            for h in range(2):
                for b in range(NB):
                    sl = h * NB + b
                    gather_wait(sl, b)
                    scat_issue(sl, b)
                for b in range(NB):
                    sl = h * NB + b
                    nsl = (1 - h) * NB + b
                    scat_wait(sl, b)
                    refill = i + h * NB + b + 2 * NB
                    nxt = i + h * NB + b + NB

                    @pl.when(refill < nch)
                    def _():
                        idx_issue(refill, sl)

                    @pl.when(nxt < nch)
                    def _():
                        idx_wait(nxt, nsl)
                        gather_issue(nsl, b)

        plsc.subcore_barrier()
        pltpu.sync_copy(acc_sh.at[pl.ds(row0, rpt)], out_hbm.at[c, pl.ds(row0, rpt)])

    return pl.kernel(
        body,
        out_type=jax.ShapeDtypeStruct((NC, n, d), jnp.float32),
        mesh=mesh,
        scratch_types=[
            [pltpu.VMEM((K,), jnp.int32) for _ in range(2 * NB)],
            [pltpu.VMEM((K,), jnp.int32) for _ in range(2 * NB)],
            [pltpu.VMEM((K, d), jnp.float32) for _ in range(NB)],
            [pltpu.SemaphoreType.DMA for _ in range(NB)],
            [pltpu.SemaphoreType.DMA for _ in range(NB)],
            [pltpu.SemaphoreType.DMA for _ in range(2 * NB)],
            pltpu.VMEM_SHARED((n, d), jnp.float32),
        ],
    )


def _sc_deg(n, nch):
    ND = 4
    assert nch % ND == 0
    rpt = n // NS
    assert rpt * NS == n and rpt % 8 == 0
    mesh = plsc.VectorSubcoreMesh(core_axis_name="c", subcore_axis_name="s", num_cores=NC, num_subcores=NS)

    def body(dst_hbm, ones_hbm, out_hbm, didx, ones_v, ssems, isems, acc_sh):
        c = lax.axis_index("c")
        s = lax.axis_index("s")
        wid = c * NS + s
        row0 = s * rpt

        @pl.loop(0, K)
        def _(i):
            ones_v[i, :] = jnp.ones((LANES,), jnp.float32)

        def idx_issue(ch, sl):
            pltpu.async_copy(dst_hbm.at[wid, ch], didx[sl], isems[sl])

        def idx_wait(ch, sl):
            pltpu.make_async_copy(dst_hbm.at[wid, ch], didx[sl], isems[sl]).wait()

        for u in range(ND):
            idx_issue(u, u)
        pltpu.sync_copy(ones_hbm.at[pl.ds(row0, rpt)], acc_sh.at[pl.ds(row0, rpt)])
        plsc.subcore_barrier()

        @pl.loop(0, nch, step=ND)
        def _(i):
            for b in range(ND):
                idx_wait(i + b, b)
                pltpu.async_copy(ones_v, acc_sh.at[didx[b]], ssems[b], add=True)
            for b in range(ND):
                pltpu.make_async_copy(ones_v, acc_sh.at[didx[b]], ssems[b]).wait()

                @pl.when(i + ND + b < nch)
                def _():
                    idx_issue(i + ND + b, b)

        plsc.subcore_barrier()
        pltpu.sync_copy(acc_sh.at[pl.ds(row0, rpt)], out_hbm.at[c, pl.ds(row0, rpt)])

    return pl.kernel(
        body,
        out_type=jax.ShapeDtypeStruct((NC, n, LANES), jnp.float32),
        mesh=mesh,
        scratch_types=[
            [pltpu.VMEM((K,), jnp.int32) for _ in range(ND)],
            pltpu.VMEM((K, LANES), jnp.float32),
            [pltpu.SemaphoreType.DMA for _ in range(ND)],
            [pltpu.SemaphoreType.DMA for _ in range(ND)],
            pltpu.VMEM_SHARED((n, LANES), jnp.float32),
        ],
    )


BN = 2000


def _dinv_blk(deg_ref):
    d = deg_ref[0, :, 0] + deg_ref[1, :, 0] - 1.0
    return lax.rsqrt(d)[:, None]


def _tc_matmul(x, w, npad):
    n, f = x.shape
    h = w.shape[1]

    def body(x_ref, w_ref, y_ref):
        y_ref[...] = jnp.dot(x_ref[...], w_ref[...],
                             preferred_element_type=jnp.float32)

    return pl.pallas_call(
        body,
        grid=(n // BN,),
        in_specs=[
            pl.BlockSpec((BN, f), lambda i: (i, 0)),
            pl.BlockSpec((f, h), lambda i: (0, 0)),
        ],
        out_specs=pl.BlockSpec((BN, h), lambda i: (i, 0)),
        out_shape=jax.ShapeDtypeStruct((npad, h), jnp.float32),
    )(x, w)


def _tc_scale(degp, xw, n):
    npad, h = xw.shape

    def body(deg_ref, xw_ref, y_ref):
        y_ref[...] = _dinv_blk(deg_ref) * xw_ref[...]

    return pl.pallas_call(
        body,
        grid=(n // BN,),
        in_specs=[
            pl.BlockSpec((NC, BN, LANES), lambda i: (0, i, 0)),
            pl.BlockSpec((BN, h), lambda i: (i, 0)),
        ],
        out_specs=pl.BlockSpec((BN, h), lambda i: (i, 0)),
        out_shape=jax.ShapeDtypeStruct((npad, h), jnp.float32),
    )(degp, xw)


def _tc_mid(degp, p, y, b, w, n):
    npad, h = y.shape
    h2 = w.shape[1]

    def body(deg_ref, p_ref, y_ref, b_ref, w_ref, out_ref):
        dinv = _dinv_blk(deg_ref)
        pre = dinv * (p_ref[0] + p_ref[1] - y_ref[...]) + b_ref[...]
        act = jnp.maximum(pre, 0.0)
        out_ref[...] = dinv * jnp.dot(
            act, w_ref[...], preferred_element_type=jnp.float32)

    return pl.pallas_call(
        body,
        grid=(n // BN,),
        in_specs=[
            pl.BlockSpec((NC, BN, LANES), lambda i: (0, i, 0)),
            pl.BlockSpec((NC, BN, h), lambda i: (0, i, 0)),
            pl.BlockSpec((BN, h), lambda i: (i, 0)),
            pl.BlockSpec((1, h), lambda i: (0, 0)),
            pl.BlockSpec((h, h2), lambda i: (0, 0)),
        ],
        out_specs=pl.BlockSpec((BN, h2), lambda i: (i, 0)),
        out_shape=jax.ShapeDtypeStruct((npad, h2), jnp.float32),
    )(degp, p, y, b, w)


def _tc_final(degp, q, y, b, wc, bc, n):
    npad, h = y.shape
    ncls = wc.shape[1]

    def body(deg_ref, q_ref, y_ref, b_ref, wc_ref, bc_ref, out_ref):
        dinv = _dinv_blk(deg_ref)
        pre = dinv * (q_ref[0] + q_ref[1] - y_ref[...]) + b_ref[...]
        act = jnp.maximum(pre, 0.0)
        logits = jnp.dot(act, wc_ref[...],
                         preferred_element_type=jnp.float32) + bc_ref[...]
        m = jnp.max(logits, axis=1, keepdims=True)
        lse = jnp.log(jnp.sum(jnp.exp(logits - m), axis=1, keepdims=True)) + m
        out_ref[...] = logits - lse

    return pl.pallas_call(
        body,
        grid=(n // BN,),
        in_specs=[
            pl.BlockSpec((NC, BN, LANES), lambda i: (0, i, 0)),
            pl.BlockSpec((NC, BN, h), lambda i: (0, i, 0)),
            pl.BlockSpec((BN, h), lambda i: (i, 0)),
            pl.BlockSpec((1, h), lambda i: (0, 0)),
            pl.BlockSpec((h, ncls), lambda i: (0, 0)),
            pl.BlockSpec((1, ncls), lambda i: (0, 0)),
        ],
        out_specs=pl.BlockSpec((BN, ncls), lambda i: (i, 0)),
        out_shape=jax.ShapeDtypeStruct((n, ncls), jnp.float32),
    )(degp, q, y, b, wc, bc)


def kernel(x, edge_idx, W1, b1, W2, b2, Wc, bc):
    n, f = x.shape
    e = edge_idx.shape[1]
    npad = _rpt(n) * NS
    src = edge_idx[0].astype(jnp.int32)
    dst = edge_idx[1].astype(jnp.int32)
    ones = jnp.ones((npad, LANES), jnp.float32)

    epw = e // NW
    epw_pad = -(-epw // (K * 2 * NB)) * (K * 2 * NB)
    nch = epw_pad // K
    padw = epw_pad - epw
    pad_rows = n + jnp.arange(NW * padw, dtype=jnp.int32).reshape(NW, padw) % (npad - n)
    src2 = jnp.concatenate([src.reshape(NW, epw), pad_rows], axis=1).reshape(NW, nch, K)
    dst2 = jnp.concatenate([dst.reshape(NW, epw), pad_rows], axis=1).reshape(NW, nch, K)

    degp = _sc_deg(npad, nch)(dst2, ones)
    xw1 = _tc_matmul(x, W1, npad)
    y1 = _tc_scale(degp, xw1, n)
    p = _sc_agg(npad, nch, W1.shape[1])(src2, dst2, y1)
    y2 = _tc_mid(degp, p, y1, b1.reshape(1, -1), W2, n)
    q = _sc_agg(npad, nch, W2.shape[1])(src2, dst2, y2)
    return _tc_final(degp, q, y2, b2.reshape(1, -1), Wc, bc.reshape(1, -1), n)
